# baseline (speedup 1.0000x reference)
import jax
import jax.numpy as jnp
from jax.experimental import pallas as pl
from jax.experimental.pallas import tpu as pltpu

_N = 10000
_D = 128
_CH = 200
_NBUF = 5
_NCHUNK = _N // _CH


def _body(modal_ref, adj_hbm, feature_ref, w_ref, b_ref, out_ref,
          buf_ref, support_ref, sem):
    i = pl.program_id(0)
    slot = jax.lax.rem(i, _NBUF)

    @pl.when(i == 0)
    def _prologue():
        for s in range(_NBUF):
            pltpu.make_async_copy(
                adj_hbm.at[pl.ds(s * _CH, _CH), :],
                buf_ref.at[s],
                sem.at[s],
            ).start()
        support_ref[:] = jnp.dot(feature_ref[:], w_ref[:],
                                 preferred_element_type=jnp.float32)

    pltpu.make_async_copy(
        adj_hbm.at[pl.ds(i * _CH, _CH), :],
        buf_ref.at[slot],
        sem.at[slot],
    ).wait()

    acc = jnp.dot(buf_ref[slot], support_ref[:],
                  preferred_element_type=jnp.float32)
    heter = acc + b_ref[:]
    feat_blk = feature_ref[pl.ds(i * _CH, _CH), :]
    out_ref[:] = jnp.where(modal_ref[0] > 1, heter, feat_blk)

    nxt = i + _NBUF

    @pl.when(nxt < _NCHUNK)
    def _refill():
        pltpu.make_async_copy(
            adj_hbm.at[pl.ds(nxt * _CH, _CH), :],
            buf_ref.at[slot],
            sem.at[slot],
        ).start()


def kernel(feature, num_modal, adj_weight, W, b):
    feature = feature.astype(jnp.float32)
    modal = jnp.asarray(num_modal, jnp.int32).reshape(1)
    b2 = b.reshape(1, _D)

    grid_spec = pltpu.PrefetchScalarGridSpec(
        num_scalar_prefetch=1,
        grid=(_NCHUNK,),
        in_specs=[
            pl.BlockSpec(memory_space=pl.ANY),
            pl.BlockSpec((_N, _D), lambda i, modal_ref: (0, 0)),
            pl.BlockSpec((_D, _D), lambda i, modal_ref: (0, 0)),
            pl.BlockSpec((1, _D), lambda i, modal_ref: (0, 0)),
        ],
        out_specs=pl.BlockSpec((_CH, _D), lambda i, modal_ref: (i, 0)),
        scratch_shapes=[
            pltpu.VMEM((_NBUF, _CH, _N), jnp.float32),
            pltpu.VMEM((_N, _D), jnp.float32),
            pltpu.SemaphoreType.DMA((_NBUF,)),
        ],
    )

    out = pl.pallas_call(
        _body,
        grid_spec=grid_spec,
        out_shape=jax.ShapeDtypeStruct((_N, _D), jnp.float32),
        compiler_params=pltpu.CompilerParams(
            dimension_semantics=("arbitrary",),
        ),
    )(modal, adj_weight, feature, W, b2)
    return out


# speedup vs baseline: 1.0130x; 1.0130x over previous
import jax
import jax.numpy as jnp
from jax.experimental import pallas as pl
from jax.experimental.pallas import tpu as pltpu

_N = 10000
_D = 128
_CH = 200
_NBUF = 4
_NCHUNK = _N // _CH


def _body(modal_ref, adj_hbm, feature_ref, w_ref, b_ref, out_ref,
          buf_ref, support_ref, sem):
    i = pl.program_id(0)
    slot = jax.lax.rem(i, _NBUF)

    @pl.when(i == 0)
    def _prologue():
        for s in range(_NBUF):
            pltpu.make_async_copy(
                adj_hbm.at[pl.ds(s * _CH, _CH), :],
                buf_ref.at[s],
                sem.at[s],
            ).start()
        support_ref[:] = jnp.dot(feature_ref[:], w_ref[:],
                                 preferred_element_type=jnp.float32)

    pltpu.make_async_copy(
        adj_hbm.at[pl.ds(i * _CH, _CH), :],
        buf_ref.at[slot],
        sem.at[slot],
    ).wait()

    acc = jnp.dot(buf_ref[slot], support_ref[:],
                  preferred_element_type=jnp.float32)
    heter = acc + b_ref[:]
    feat_blk = feature_ref[pl.ds(i * _CH, _CH), :]
    out_ref[:] = jnp.where(modal_ref[0] > 1, heter, feat_blk)

    nxt = i + _NBUF

    @pl.when(nxt < _NCHUNK)
    def _refill():
        pltpu.make_async_copy(
            adj_hbm.at[pl.ds(nxt * _CH, _CH), :],
            buf_ref.at[slot],
            sem.at[slot],
        ).start()


def kernel(feature, num_modal, adj_weight, W, b):
    feature = feature.astype(jnp.float32)
    modal = jnp.asarray(num_modal, jnp.int32).reshape(1)
    b2 = b.reshape(1, _D)

    grid_spec = pltpu.PrefetchScalarGridSpec(
        num_scalar_prefetch=1,
        grid=(_NCHUNK,),
        in_specs=[
            pl.BlockSpec(memory_space=pl.ANY),
            pl.BlockSpec((_N, _D), lambda i, modal_ref: (0, 0)),
            pl.BlockSpec((_D, _D), lambda i, modal_ref: (0, 0)),
            pl.BlockSpec((1, _D), lambda i, modal_ref: (0, 0)),
        ],
        out_specs=pl.BlockSpec((_CH, _D), lambda i, modal_ref: (i, 0)),
        scratch_shapes=[
            pltpu.VMEM((_NBUF, _CH, _N), jnp.float32),
            pltpu.VMEM((_N, _D), jnp.float32),
            pltpu.SemaphoreType.DMA((_NBUF,)),
        ],
    )

    out = pl.pallas_call(
        _body,
        grid_spec=grid_spec,
        out_shape=jax.ShapeDtypeStruct((_N, _D), jnp.float32),
        compiler_params=pltpu.CompilerParams(
            dimension_semantics=("arbitrary",),
        ),
    )(modal, adj_weight, feature, W, b2)
    return out


# speedup vs baseline: 1.0406x; 1.0272x over previous
import jax
import jax.numpy as jnp
from jax.experimental import pallas as pl
from jax.experimental.pallas import tpu as pltpu

_N = 10000
_D = 128
_CH = 200
_NBUF = 2
_NCHUNK = _N // _CH


def _body(modal_ref, adj_hbm, feature_ref, w_ref, b_ref, out_ref,
          buf_ref, support_ref, sem):
    i = pl.program_id(0)
    slot = jax.lax.rem(i, _NBUF)

    @pl.when(i == 0)
    def _prologue():
        for s in range(_NBUF):
            pltpu.make_async_copy(
                adj_hbm.at[pl.ds(s * _CH, _CH), :],
                buf_ref.at[s],
                sem.at[s],
            ).start()
        support_ref[:] = jnp.dot(feature_ref[:], w_ref[:],
                                 preferred_element_type=jnp.float32)

    pltpu.make_async_copy(
        adj_hbm.at[pl.ds(i * _CH, _CH), :],
        buf_ref.at[slot],
        sem.at[slot],
    ).wait()

    acc = jnp.dot(buf_ref[slot], support_ref[:],
                  preferred_element_type=jnp.float32)
    heter = acc + b_ref[:]
    feat_blk = feature_ref[pl.ds(i * _CH, _CH), :]
    out_ref[:] = jnp.where(modal_ref[0] > 1, heter, feat_blk)

    nxt = i + _NBUF

    @pl.when(nxt < _NCHUNK)
    def _refill():
        pltpu.make_async_copy(
            adj_hbm.at[pl.ds(nxt * _CH, _CH), :],
            buf_ref.at[slot],
            sem.at[slot],
        ).start()


def kernel(feature, num_modal, adj_weight, W, b):
    feature = feature.astype(jnp.float32)
    modal = jnp.asarray(num_modal, jnp.int32).reshape(1)
    b2 = b.reshape(1, _D)

    grid_spec = pltpu.PrefetchScalarGridSpec(
        num_scalar_prefetch=1,
        grid=(_NCHUNK,),
        in_specs=[
            pl.BlockSpec(memory_space=pl.ANY),
            pl.BlockSpec((_N, _D), lambda i, modal_ref: (0, 0)),
            pl.BlockSpec((_D, _D), lambda i, modal_ref: (0, 0)),
            pl.BlockSpec((1, _D), lambda i, modal_ref: (0, 0)),
        ],
        out_specs=pl.BlockSpec((_CH, _D), lambda i, modal_ref: (i, 0)),
        scratch_shapes=[
            pltpu.VMEM((_NBUF, _CH, _N), jnp.float32),
            pltpu.VMEM((_N, _D), jnp.float32),
            pltpu.SemaphoreType.DMA((_NBUF,)),
        ],
    )

    out = pl.pallas_call(
        _body,
        grid_spec=grid_spec,
        out_shape=jax.ShapeDtypeStruct((_N, _D), jnp.float32),
        compiler_params=pltpu.CompilerParams(
            dimension_semantics=("arbitrary",),
        ),
    )(modal, adj_weight, feature, W, b2)
    return out
